# baseline (speedup 1.0000x reference)
import functools

import jax
import jax.numpy as jnp
import numpy as np
from jax.experimental import pallas as pl
from jax.experimental.pallas import tpu as pltpu

_N = 10000
_E = 160000
_CS = 8
_CV = 8
_CIN = _CS + _CV
_O = 8
_H = 32
_B = 32
_L = 2
_OS = 3
_OV = 3

_BN = 400
_BE = 2000


def _fib_sphere(n):
    i = np.arange(n, dtype=np.float64)
    golden = np.pi * (3.0 - np.sqrt(5.0))
    y = 1.0 - 2.0 * (i + 0.5) / n
    r = np.sqrt(np.maximum(1.0 - y * y, 0.0))
    th = golden * i
    pts = np.stack([r * np.cos(th), y, r * np.sin(th)], axis=-1)
    return np.asarray(pts, dtype=np.float32)


def _embed_body(x_ref, w_ref, b_ref, out_ref):
    x = x_ref[...]
    h = x.reshape(-1, _CIN) @ w_ref[...] + b_ref[...][None, :]
    out_ref[...] = h.reshape(-1, _O, _H)


def _edge_msg_body(invs_ref, d2_ref, g_ref, ws1_ref, bs1_ref, ws2_ref,
                   bs2_ref, out_ref):
    invs = invs_ref[...]
    inv1 = invs[:, :, 0:1]
    inv2 = invs[:, :, 1:2]
    ws1 = ws1_ref[...]
    pre = (inv1 * ws1[0] + inv2 * ws1[1]
           + bs1_ref[...][None, None, :])
    act = jax.nn.gelu(pre).reshape(-1, _B)
    k = act @ ws2_ref[...] + bs2_ref[...][None, :]
    window = jnp.exp(-0.5 * d2_ref[...])
    out_ref[...] = k.reshape(-1, _O, _H) * window * g_ref[...]


def _dense_layer_body(h_ref, agg_ref, kor_ref, wl_ref, wf1_ref, bf1_ref,
                      wf2_ref, bf2_ref, out_ref):
    h = h_ref[...]
    agg = agg_ref[...]
    kor = kor_ref[...]
    y = agg[:, 0:1, :] * kor[:, 0, :]
    for p in range(1, _O):
        y = y + agg[:, p:p + 1, :] * kor[:, p, :]
    y = (y.reshape(-1, _H) @ wl_ref[...]).reshape(-1, _O, _H)
    h = h + y
    m = jnp.mean(h, axis=-1, keepdims=True)
    v = jnp.mean((h - m) * (h - m), axis=-1, keepdims=True)
    z = (h - m) * jax.lax.rsqrt(v + 1e-5)
    zf = z.reshape(-1, _H)
    z2 = jax.nn.gelu(zf @ wf1_ref[...] + bf1_ref[...][None, :])
    z2 = z2 @ wf2_ref[...] + bf2_ref[...][None, :]
    out_ref[...] = h + z2.reshape(-1, _O, _H)


def _readout_body(h_ref, wo_ref, bo_ref, out_ref):
    h = h_ref[...]
    outp = h.reshape(-1, _H) @ wo_ref[...] + bo_ref[...][None, :]
    out_ref[...] = outp.reshape(-1, _O, _OS + _OV)


def _full_spec(shape):
    return pl.BlockSpec(shape, lambda i: tuple(0 for _ in shape))


def kernel(scalar, vector, pos, edge_index, w_embed, b_embed, ws1, bs1, ws2,
           bs2, wo1, bo1, wo2, bo2, wl, wf1, bf1, wf2, bf2, w_out, b_out):
    ori = jnp.asarray(_fib_sphere(_O))

    n_blocks = _N // _BN
    e_blocks = _E // _BE

    x_s = jnp.broadcast_to(scalar[:, None, :], (_N, _O, _CS))
    x_v = jnp.einsum('nvd,od->nov', vector, ori)
    x = jnp.concatenate([x_s, x_v], axis=-1)
    h = pl.pallas_call(
        _embed_body,
        grid=(n_blocks,),
        in_specs=[
            pl.BlockSpec((_BN, _O, _CIN), lambda i: (i, 0, 0)),
            _full_spec((_CIN, _H)),
            _full_spec((_H,)),
        ],
        out_specs=pl.BlockSpec((_BN, _O, _H), lambda i: (i, 0, 0)),
        out_shape=jax.ShapeDtypeStruct((_N, _O, _H), jnp.float32),
    )(x, w_embed, b_embed)

    src = edge_index[0]
    dst = edge_index[1]
    rel = pos[src] - pos[dst]
    inv1 = rel @ ori.T
    d2 = jnp.sum(rel * rel, axis=-1)
    inv2 = jnp.sqrt(jnp.maximum(d2[:, None] - inv1 * inv1, 0.0) + 1e-8)
    invs = jnp.stack([inv1, inv2], axis=-1)
    d2_3 = d2[:, None, None]

    ori_inv = (ori @ ori.T)[..., None]

    edge_call = pl.pallas_call(
        _edge_msg_body,
        grid=(e_blocks,),
        in_specs=[
            pl.BlockSpec((_BE, _O, 2), lambda i: (i, 0, 0)),
            pl.BlockSpec((_BE, 1, 1), lambda i: (i, 0, 0)),
            pl.BlockSpec((_BE, _O, _H), lambda i: (i, 0, 0)),
            _full_spec((2, _B)),
            _full_spec((_B,)),
            _full_spec((_B, _H)),
            _full_spec((_H,)),
        ],
        out_specs=pl.BlockSpec((_BE, _O, _H), lambda i: (i, 0, 0)),
        out_shape=jax.ShapeDtypeStruct((_E, _O, _H), jnp.float32),
    )

    dense_call = pl.pallas_call(
        _dense_layer_body,
        grid=(n_blocks,),
        in_specs=[
            pl.BlockSpec((_BN, _O, _H), lambda i: (i, 0, 0)),
            pl.BlockSpec((_BN, _O, _H), lambda i: (i, 0, 0)),
            _full_spec((_O, _O, _H)),
            _full_spec((_H, _H)),
            _full_spec((_H, 4 * _H)),
            _full_spec((4 * _H,)),
            _full_spec((4 * _H, _H)),
            _full_spec((_H,)),
        ],
        out_specs=pl.BlockSpec((_BN, _O, _H), lambda i: (i, 0, 0)),
        out_shape=jax.ShapeDtypeStruct((_N, _O, _H), jnp.float32),
    )

    for l in range(_L):
        k_or = jax.nn.gelu(ori_inv @ wo1[l] + bo1[l]) @ wo2[l] + bo2[l]
        g = jnp.take(h, src, axis=0)
        msg = edge_call(invs, d2_3, g, ws1[l], bs1[l], ws2[l], bs2[l])
        agg = jax.ops.segment_sum(msg, dst, num_segments=_N)
        h = dense_call(h, agg, k_or, wl[l], wf1[l], bf1[l], wf2[l], bf2[l])

    outp = pl.pallas_call(
        _readout_body,
        grid=(n_blocks,),
        in_specs=[
            pl.BlockSpec((_BN, _O, _H), lambda i: (i, 0, 0)),
            _full_spec((_H, _OS + _OV)),
            _full_spec((_OS + _OV,)),
        ],
        out_specs=pl.BlockSpec((_BN, _O, _OS + _OV), lambda i: (i, 0, 0)),
        out_shape=jax.ShapeDtypeStruct((_N, _O, _OS + _OV), jnp.float32),
    )(h, w_out, b_out)

    out_s = jnp.mean(outp[..., :_OS], axis=-2)
    out_v = jnp.einsum('noc,od->ncd', outp[..., _OS:], ori) / _O
    hidden = jnp.mean(h, axis=-2)
    out = out_v * out_s[..., None]
    return out.reshape(-1, 3), hidden.reshape(-1, _H)
